# baseline (speedup 1.0000x reference)
import functools

import jax
import jax.numpy as jnp
from jax import lax
from jax.experimental import pallas as pl
from jax.experimental.pallas import tpu as pltpu
from jax.experimental.pallas import tpu_sc as plsc

_NUM_CORES = 2
_NUM_SUBCORES = 16
_NUM_WORKERS = _NUM_CORES * _NUM_SUBCORES

_CHUNK = 128


@functools.lru_cache(maxsize=None)
def _make_gather(V, D, B):
    assert B % (_NUM_WORKERS * _CHUNK) == 0
    b_per_w = B // _NUM_WORKERS
    n_chunks = b_per_w // _CHUNK

    mesh = plsc.VectorSubcoreMesh(
        core_axis_name="c",
        subcore_axis_name="s",
        num_cores=_NUM_CORES,
        num_subcores=_NUM_SUBCORES,
    )

    @functools.partial(
        pl.kernel,
        mesh=mesh,
        out_type=jax.ShapeDtypeStruct((B, D), jnp.float32),
        scratch_types=[
            pltpu.VMEM((_CHUNK,), jnp.int32),
            pltpu.VMEM((_CHUNK, D), jnp.float32),
            pltpu.SemaphoreType.DMA,
        ],
        compiler_params=pltpu.CompilerParams(use_tc_tiling_on_sc=False),
    )
    def gather_kernel(idx_hbm, table_hbm, out_hbm, idx_v, rows_v, sem):
        wid = lax.axis_index("s") * _NUM_CORES + lax.axis_index("c")
        base = wid * b_per_w

        def body(g, carry):
            off = base + g * _CHUNK
            pltpu.sync_copy(idx_hbm.at[pl.ds(off, _CHUNK)], idx_v)
            pltpu.async_copy(table_hbm.at[idx_v], rows_v, sem).wait()
            pltpu.sync_copy(rows_v, out_hbm.at[pl.ds(off, _CHUNK)])
            return carry

        lax.fori_loop(0, n_chunks, body, 0)

    return gather_kernel


def kernel(x, table):
    batch, hist = x.shape
    vocab, dim = table.shape
    flat_idx = x.reshape(batch * hist).astype(jnp.int32)
    out = _make_gather(vocab, dim, batch * hist)(flat_idx, table)
    return out.reshape(batch, hist, dim)


# speedup vs baseline: 1.1384x; 1.1384x over previous
import functools

import jax
import jax.numpy as jnp
from jax import lax
from jax.experimental import pallas as pl
from jax.experimental.pallas import tpu as pltpu
from jax.experimental.pallas import tpu_sc as plsc

_NUM_CORES = 2
_NUM_SUBCORES = 16
_NUM_WORKERS = _NUM_CORES * _NUM_SUBCORES

_IDX_PER_STREAM = 128
_STREAMS_PER_ITER = 5
_ROWS_PER_ITER = _IDX_PER_STREAM * _STREAMS_PER_ITER
_NBUF = 4


@functools.lru_cache(maxsize=None)
def _make_gather(V, D, B):
    b_per_w = B // _NUM_WORKERS
    n_outer = b_per_w // _ROWS_PER_ITER
    assert b_per_w % _ROWS_PER_ITER == 0 and n_outer % _NBUF == 0

    mesh = plsc.VectorSubcoreMesh(
        core_axis_name="c",
        subcore_axis_name="s",
        num_cores=_NUM_CORES,
        num_subcores=_NUM_SUBCORES,
    )

    @functools.partial(
        pl.kernel,
        mesh=mesh,
        out_type=jax.ShapeDtypeStruct((B, D), jnp.float32),
        scratch_types=[
            pltpu.VMEM((b_per_w,), jnp.int32),
            [pltpu.VMEM((_ROWS_PER_ITER, D), jnp.float32) for _ in range(_NBUF)],
            [pltpu.SemaphoreType.DMA for _ in range(_NBUF)],
            [pltpu.SemaphoreType.DMA for _ in range(_NBUF)],
        ],
        compiler_params=pltpu.CompilerParams(use_tc_tiling_on_sc=False),
    )
    def gather_kernel(idx_hbm, table_hbm, out_hbm, idx_v, bufs, gsems, wsems):
        wid = lax.axis_index("s") * _NUM_CORES + lax.axis_index("c")
        base = wid * b_per_w

        pltpu.sync_copy(idx_hbm.at[pl.ds(base, b_per_w)], idx_v)

        def gather_descs(i, rows, gsem):
            descs = []
            for k in range(_STREAMS_PER_ITER):
                off = i * _ROWS_PER_ITER + k * _IDX_PER_STREAM
                descs.append(pltpu.make_async_copy(
                    table_hbm.at[idx_v.at[pl.ds(off, _IDX_PER_STREAM)]],
                    rows.at[pl.ds(k * _IDX_PER_STREAM, _IDX_PER_STREAM)],
                    gsem,
                ))
            return descs

        def wb_desc(i, rows, wsem):
            return pltpu.make_async_copy(
                rows, out_hbm.at[pl.ds(base + i * _ROWS_PER_ITER, _ROWS_PER_ITER)], wsem)

        def fire_gathers(i, rows, gsem):
            for d in gather_descs(i, rows, gsem):
                d.start()

        for b in range(_NBUF):
            fire_gathers(jnp.int32(b), bufs[b], gsems[b])

        @pl.loop(0, n_outer, step=_NBUF)
        def _outer(i0):
            for b in range(_NBUF):
                i = i0 + b
                for d in gather_descs(i, bufs[b], gsems[b]):
                    d.wait()
                wb_desc(i, bufs[b], wsems[b]).start()

                @pl.when(i < n_outer - _NBUF)
                def _refill():
                    wb_desc(i, bufs[b], wsems[b]).wait()
                    fire_gathers(i + _NBUF, bufs[b], gsems[b])

        for b in range(_NBUF):
            i_last = n_outer - _NBUF + b
            wb_desc(jnp.int32(i_last), bufs[b], wsems[b]).wait()

    return gather_kernel


def kernel(x, table):
    batch, hist = x.shape
    vocab, dim = table.shape
    flat_idx = x.reshape(batch * hist).astype(jnp.int32)
    out = _make_gather(vocab, dim, batch * hist)(flat_idx, table)
    return out.reshape(batch, hist, dim)


# speedup vs baseline: 1.4373x; 1.2626x over previous
import functools

import jax
import jax.numpy as jnp
from jax import lax
from jax.experimental import pallas as pl
from jax.experimental.pallas import tpu as pltpu
from jax.experimental.pallas import tpu_sc as plsc

_NUM_CORES = 2
_NUM_SUBCORES = 16
_NUM_WORKERS = _NUM_CORES * _NUM_SUBCORES

_BB = 128
_L = 16
_NBUF = 2


@functools.lru_cache(maxsize=None)
def _make_gather(V, D, B, H):
    assert D % 8 == 0 and B % (_NUM_WORKERS * _BB) == 0
    td_n = D // 8
    tb_per_w = B // (_NUM_WORKERS * _BB)
    n_blocks = H * tb_per_w
    assert n_blocks % _NBUF == 0

    mesh = plsc.VectorSubcoreMesh(
        core_axis_name="c",
        subcore_axis_name="s",
        num_cores=_NUM_CORES,
        num_subcores=_NUM_SUBCORES,
    )

    @functools.partial(
        pl.kernel,
        mesh=mesh,
        out_type=jax.ShapeDtypeStruct((H, td_n, B // _BB, 8, _BB), jnp.float32),
        scratch_types=[
            [pltpu.VMEM((_BB,), jnp.int32) for _ in range(_NBUF)],
            [pltpu.VMEM((_BB, D), jnp.float32) for _ in range(_NBUF)],
            [pltpu.VMEM((td_n, 8, _BB), jnp.float32) for _ in range(_NBUF)],
            [pltpu.SemaphoreType.DMA for _ in range(_NBUF)],
            [pltpu.SemaphoreType.DMA for _ in range(_NBUF)],
        ],
        compiler_params=pltpu.CompilerParams(
            use_tc_tiling_on_sc=False, needs_layout_passes=False),
    )
    def gather_kernel(xt_hbm, table_hbm, out_hbm, idxs, rows, touts, gsems, wsems):
        wid = lax.axis_index("s") * _NUM_CORES + lax.axis_index("c")
        tb_base = wid * tb_per_w

        def coords(i):
            return i // tb_per_w, tb_base + i % tb_per_w

        def stage_idx_and_fire(i, p):
            h, tb = coords(i)
            pltpu.sync_copy(xt_hbm.at[h, pl.ds(tb * _BB, _BB)], idxs[p])
            pltpu.async_copy(table_hbm.at[idxs[p]], rows[p], gsems[p])

        def drain_gather(p):
            pltpu.make_async_copy(table_hbm.at[idxs[p]], rows[p], gsems[p]).wait()

        def wb_descs(i, p):
            h, tb = coords(i)
            return [
                pltpu.make_async_copy(
                    touts[p].at[td], out_hbm.at[h, td, tb], wsems[p])
                for td in range(td_n)
            ]

        def transpose_block(p):
            for td in range(td_n):
                for d8 in range(8):
                    d_idx = jnp.full((_L,), td * 8 + d8, jnp.int32)
                    for g in range(_BB // _L):
                        b_idx = g * _L + lax.iota(jnp.int32, _L)
                        v = plsc.load_gather(rows[p], [b_idx, d_idx])
                        touts[p][td, d8, pl.ds(g * _L, _L)] = v

        for p in range(_NBUF):
            stage_idx_and_fire(jnp.int32(p), p)

        @pl.loop(0, n_blocks, step=_NBUF)
        def _blocks(i0):
            for p in range(_NBUF):
                i = i0 + p
                drain_gather(p)

                @pl.when(i >= _NBUF)
                def _wait_prev_wb():
                    for d in wb_descs(i - _NBUF, p):
                        d.wait()

                transpose_block(p)
                for d in wb_descs(i, p):
                    d.start()

                @pl.when(i + _NBUF < n_blocks)
                def _next():
                    stage_idx_and_fire(i + _NBUF, p)

        for p in range(_NBUF):
            for d in wb_descs(jnp.int32(n_blocks - _NBUF + p), p):
                d.wait()

    return gather_kernel


def kernel(x, table):
    batch, hist = x.shape
    vocab, dim = table.shape
    xt = x.T.astype(jnp.int32)
    out5 = _make_gather(vocab, dim, batch, hist)(xt, table)
    return out5.transpose(2, 4, 0, 1, 3).reshape(batch, hist, dim)


# speedup vs baseline: 1.9766x; 1.3752x over previous
import functools

import jax
import jax.numpy as jnp
from jax import lax
from jax.experimental import pallas as pl
from jax.experimental.pallas import tpu as pltpu
from jax.experimental.pallas import tpu_sc as plsc

_NUM_CORES = 2
_NUM_SUBCORES = 16
_NUM_WORKERS = _NUM_CORES * _NUM_SUBCORES

_BB = 128
_L = 16
_NBUF = 4


@functools.lru_cache(maxsize=None)
def _make_gather(V, D, B, H):
    assert D % _L == 0 and B % (_NUM_WORKERS * _BB) == 0
    td_n = D // 8
    tb_per_w = B // (_NUM_WORKERS * _BB)
    n_blocks = H * tb_per_w
    assert n_blocks % _NBUF == 0

    mesh = plsc.VectorSubcoreMesh(
        core_axis_name="c",
        subcore_axis_name="s",
        num_cores=_NUM_CORES,
        num_subcores=_NUM_SUBCORES,
    )

    @functools.partial(
        pl.kernel,
        mesh=mesh,
        out_type=jax.ShapeDtypeStruct((H, td_n, B // _BB, 8 * _BB), jnp.float32),
        scratch_types=[
            [pltpu.VMEM((_BB,), jnp.int32) for _ in range(_NBUF)],
            [pltpu.VMEM((_BB, D), jnp.float32) for _ in range(_NBUF)],
            [pltpu.VMEM((D * _BB,), jnp.float32) for _ in range(_NBUF)],
            [pltpu.SemaphoreType.DMA for _ in range(_NBUF)],
            [pltpu.SemaphoreType.DMA for _ in range(_NBUF)],
        ],
        compiler_params=pltpu.CompilerParams(
            use_tc_tiling_on_sc=False, needs_layout_passes=False),
    )
    def gather_kernel(xt_hbm, table_hbm, out_hbm, idxs, rows, touts, gsems, wsems):
        wid = lax.axis_index("s") * _NUM_CORES + lax.axis_index("c")
        tb_base = wid * tb_per_w

        def coords(i):
            return i // tb_per_w, tb_base + i % tb_per_w

        def stage_idx_and_fire(i, p):
            h, tb = coords(i)
            pltpu.sync_copy(xt_hbm.at[h, pl.ds(tb * _BB, _BB)], idxs[p])
            pltpu.async_copy(table_hbm.at[idxs[p]], rows[p], gsems[p])

        def drain_gather(p):
            pltpu.make_async_copy(table_hbm.at[idxs[p]], rows[p], gsems[p]).wait()

        def wb_descs(i, p):
            h, tb = coords(i)
            return [
                pltpu.make_async_copy(
                    touts[p].at[pl.ds(td * 8 * _BB, 8 * _BB)],
                    out_hbm.at[h, td, tb],
                    wsems[p],
                )
                for td in range(td_n)
            ]

        scatter_bases = [
            (v * _L + lax.iota(jnp.int32, _L)) * _BB for v in range(D // _L)
        ]

        def transpose_block(p):
            @plsc.parallel_loop(0, _BB, step=1, unroll=8)
            def _t(b):
                for v in range(D // _L):
                    vec = rows[p][b, pl.ds(v * _L, _L)]
                    plsc.store_scatter(touts[p], [scatter_bases[v] + b], vec)

        for p in range(_NBUF):
            stage_idx_and_fire(jnp.int32(p), p)

        @pl.loop(0, n_blocks, step=_NBUF)
        def _blocks(i0):
            for p in range(_NBUF):
                i = i0 + p
                drain_gather(p)

                @pl.when(i >= _NBUF)
                def _wait_prev_wb():
                    for d in wb_descs(i - _NBUF, p):
                        d.wait()

                transpose_block(p)
                for d in wb_descs(i, p):
                    d.start()

                @pl.when(i + _NBUF < n_blocks)
                def _next():
                    stage_idx_and_fire(i + _NBUF, p)

        for p in range(_NBUF):
            for d in wb_descs(jnp.int32(n_blocks - _NBUF + p), p):
                d.wait()

    return gather_kernel


def kernel(x, table):
    batch, hist = x.shape
    vocab, dim = table.shape
    xt = x.T.astype(jnp.int32)
    out5 = _make_gather(vocab, dim, batch, hist)(xt, table)
    out5 = out5.reshape(hist, dim // 8, batch // _BB, 8, _BB)
    return out5.transpose(2, 4, 0, 1, 3).reshape(batch, hist, dim)


# speedup vs baseline: 2.2015x; 1.1138x over previous
import functools

import jax
import jax.numpy as jnp
from jax import lax
from jax.experimental import pallas as pl
from jax.experimental.pallas import tpu as pltpu
from jax.experimental.pallas import tpu_sc as plsc

_NUM_CORES = 2
_NUM_SUBCORES = 16
_NUM_WORKERS = _NUM_CORES * _NUM_SUBCORES

_BB = 128
_L = 16
_NBUF = 4


@functools.lru_cache(maxsize=None)
def _make_gather(V, D, B, H):
    assert D % _L == 0 and B % (_NUM_WORKERS * _BB) == 0
    td_n = D // 8
    tb_per_w = B // (_NUM_WORKERS * _BB)
    b_per_w = tb_per_w * _BB
    n_blocks = H * tb_per_w
    assert n_blocks % _NBUF == 0

    mesh = plsc.VectorSubcoreMesh(
        core_axis_name="c",
        subcore_axis_name="s",
        num_cores=_NUM_CORES,
        num_subcores=_NUM_SUBCORES,
    )

    @functools.partial(
        pl.kernel,
        mesh=mesh,
        out_type=jax.ShapeDtypeStruct((H, td_n, B // _BB, 8 * _BB), jnp.float32),
        scratch_types=[
            pltpu.VMEM((H, b_per_w), jnp.int32),
            [pltpu.VMEM((_BB, D), jnp.float32) for _ in range(_NBUF)],
            [pltpu.VMEM((D * _BB,), jnp.float32) for _ in range(_NBUF)],
            [pltpu.SemaphoreType.DMA for _ in range(_NBUF)],
            [pltpu.SemaphoreType.DMA for _ in range(_NBUF)],
        ],
        compiler_params=pltpu.CompilerParams(
            use_tc_tiling_on_sc=False, needs_layout_passes=False),
    )
    def gather_kernel(xt_hbm, table_hbm, out_hbm, idx_all, rows, touts, gsems, wsems):
        wid = lax.axis_index("s") * _NUM_CORES + lax.axis_index("c")
        tb_base = wid * tb_per_w

        pltpu.sync_copy(xt_hbm.at[:, pl.ds(wid * b_per_w, b_per_w)], idx_all)

        def coords(i):
            return i // tb_per_w, i % tb_per_w

        def gather_desc(i, p):
            h, tbl = coords(i)
            return pltpu.make_async_copy(
                table_hbm.at[idx_all.at[h, pl.ds(tbl * _BB, _BB)]],
                rows[p],
                gsems[p],
            )

        def wb_descs(i, p):
            h, tbl = coords(i)
            return [
                pltpu.make_async_copy(
                    touts[p].at[pl.ds(td * 8 * _BB, 8 * _BB)],
                    out_hbm.at[h, td, tb_base + tbl],
                    wsems[p],
                )
                for td in range(td_n)
            ]

        scatter_bases = [
            (v * _L + lax.iota(jnp.int32, _L)) * _BB for v in range(D // _L)
        ]

        def transpose_block(p):
            @plsc.parallel_loop(0, _BB, step=1, unroll=8)
            def _t(b):
                for v in range(D // _L):
                    vec = rows[p][b, pl.ds(v * _L, _L)]
                    plsc.store_scatter(touts[p], [scatter_bases[v] + b], vec)

        for p in range(_NBUF):
            gather_desc(jnp.int32(p), p).start()

        @pl.loop(0, n_blocks, step=_NBUF)
        def _blocks(i0):
            for p in range(_NBUF):
                i = i0 + p
                gather_desc(i, p).wait()

                @pl.when(i >= _NBUF)
                def _wait_prev_wb():
                    for d in wb_descs(i - _NBUF, p):
                        d.wait()

                transpose_block(p)
                for d in wb_descs(i, p):
                    d.start()

                @pl.when(i + _NBUF < n_blocks)
                def _next():
                    gather_desc(i + _NBUF, p).start()

        for p in range(_NBUF):
            for d in wb_descs(jnp.int32(n_blocks - _NBUF + p), p):
                d.wait()

    return gather_kernel


def kernel(x, table):
    batch, hist = x.shape
    vocab, dim = table.shape
    xt = x.T.astype(jnp.int32)
    out5 = _make_gather(vocab, dim, batch, hist)(xt, table)
    out5 = out5.reshape(hist, dim // 8, batch // _BB, 8, _BB)
    return out5.transpose(2, 4, 0, 1, 3).reshape(batch, hist, dim)
